# baseline (speedup 1.0000x reference)
import functools

import jax
import jax.numpy as jnp
from jax import lax
from jax.experimental import pallas as pl
from jax.experimental.pallas import tpu as pltpu
from jax.experimental.pallas import tpu_sc as plsc

_CHUNK_ROWS = 16
_NBUF = 7


def kernel(x, emb_weight):
    seq_len = x.shape[1]
    dim = emb_weight.shape[1]
    info = plsc.get_sparse_core_info()
    num_cores = info.num_cores
    ns = info.num_subcores
    nw = num_cores * ns
    rows_per_w = seq_len // nw
    nchunks = rows_per_w // _CHUNK_ROWS
    mesh = plsc.VectorSubcoreMesh(
        core_axis_name="c", subcore_axis_name="s", num_cores=num_cores
    )

    @functools.partial(
        pl.kernel,
        mesh=mesh,
        out_type=jax.ShapeDtypeStruct((seq_len, dim), emb_weight.dtype),
        scratch_types=[
            pltpu.VMEM((_NBUF, _CHUNK_ROWS, dim), emb_weight.dtype),
        ]
        + [pltpu.SemaphoreType.DMA] * (2 * _NBUF),
    )
    def copy_k(table_hbm, out_hbm, buf, *sems):
        wid = lax.axis_index("s") * num_cores + lax.axis_index("c")
        base = wid * rows_per_w
        in_sems = sems[:_NBUF]
        out_sems = sems[_NBUF:]

        def gather(i):
            return pltpu.async_copy(
                table_hbm.at[pl.ds(base + i * _CHUNK_ROWS, _CHUNK_ROWS)],
                buf.at[i % _NBUF],
                in_sems[i % _NBUF],
            )

        def scatter(i):
            return pltpu.async_copy(
                buf.at[i % _NBUF],
                out_hbm.at[pl.ds(base + i * _CHUNK_ROWS, _CHUNK_ROWS)],
                out_sems[i % _NBUF],
            )

        gathers = [None] * nchunks
        scatters = [None] * nchunks
        for i in range(min(_NBUF - 1, nchunks)):
            gathers[i] = gather(i)
        for i in range(nchunks):
            j = i + _NBUF - 1
            if j < nchunks:
                if j >= _NBUF:
                    scatters[j - _NBUF].wait()
                gathers[j] = gather(j)
            gathers[i].wait()
            scatters[i] = scatter(i)
        for i in range(max(0, nchunks - _NBUF), nchunks):
            scatters[i].wait()

    return copy_k(emb_weight)[None, :, :]


# speedup vs baseline: 1.0070x; 1.0070x over previous
import functools

import jax
from jax import lax
from jax.experimental import pallas as pl
from jax.experimental.pallas import tpu as pltpu
from jax.experimental.pallas import tpu_sc as plsc

_CHUNK_ROWS = 16
_NBUF = 7


def kernel(x, emb_weight):
    seq_len = x.shape[1]
    dim = emb_weight.shape[1]
    info = plsc.get_sparse_core_info()
    num_cores = info.num_cores
    ns = info.num_subcores
    nw = num_cores * ns
    rows_per_w = seq_len // nw
    nchunks = rows_per_w // _CHUNK_ROWS
    mesh = plsc.VectorSubcoreMesh(
        core_axis_name="c", subcore_axis_name="s", num_cores=num_cores
    )

    @functools.partial(
        pl.kernel,
        mesh=mesh,
        out_type=jax.ShapeDtypeStruct((seq_len, dim), emb_weight.dtype),
        scratch_types=[
            pltpu.VMEM((_NBUF, _CHUNK_ROWS, dim), emb_weight.dtype),
        ]
        + [pltpu.SemaphoreType.DMA] * (2 * _NBUF),
    )
    def copy_k(table_hbm, out_hbm, buf, *sems):
        wid = lax.axis_index("s") * num_cores + lax.axis_index("c")
        base = wid * rows_per_w
        in_sems = sems[:_NBUF]
        out_sems = sems[_NBUF:]

        def gather(i):
            return pltpu.async_copy(
                table_hbm.at[pl.ds(base + i * _CHUNK_ROWS, _CHUNK_ROWS)],
                buf.at[i % _NBUF],
                in_sems[i % _NBUF],
            )

        def scatter(i):
            return pltpu.async_copy(
                buf.at[i % _NBUF],
                out_hbm.at[pl.ds(base + i * _CHUNK_ROWS, _CHUNK_ROWS)],
                out_sems[i % _NBUF],
            )

        gathers = [None] * nchunks
        scatters = [None] * nchunks
        for i in range(min(_NBUF - 1, nchunks)):
            gathers[i] = gather(i)
        for i in range(nchunks):
            j = i + _NBUF - 1
            if j < nchunks:
                if j >= _NBUF:
                    scatters[j - _NBUF].wait()
                gathers[j] = gather(j)
            gathers[i].wait()
            scatters[i] = scatter(i)
        for i in range(max(0, nchunks - _NBUF), nchunks):
            scatters[i].wait()

    return copy_k(emb_weight)[None, :, :]
